# baseline (speedup 1.0000x reference)
import functools

import jax
import jax.numpy as jnp
from jax import lax
from jax.experimental import pallas as pl
from jax.experimental.pallas import tpu as pltpu
from jax.experimental.pallas import tpu_sc as plsc

_NC = 2
_NS = 16
_NW = _NC * _NS
_CHUNK = 128
_L = 16


def _silu(x):
    return x * jax.nn.sigmoid(x)


def _bdot(a, b):
    return jnp.dot(a, b, preferred_element_type=jnp.float32)


def _layernorm(x, g, b):
    mu = jnp.mean(x, axis=-1, keepdims=True)
    var = jnp.mean((x - mu) ** 2, axis=-1, keepdims=True)
    return (x - mu) / jnp.sqrt(var + 1e-5) * g + b


def _row_sel(be, r):
    return (lax.broadcasted_iota(jnp.int32, (be, r), 0) // 128
            == lax.broadcasted_iota(jnp.int32, (be, r), 1)).astype(jnp.float32)


def _lane_sel(be):
    return (lax.broadcasted_iota(jnp.int32, (be, 128), 0) % 128
            == lax.broadcasted_iota(jnp.int32, (be, 128), 1)).astype(jnp.float32)


def _unpack_cols(p, be):
    r = be // 128
    spread = jax.lax.dot_general(_row_sel(be, r), p, (((1,), (0,)), ((), ())),
                                 preferred_element_type=jnp.float32)
    return jnp.sum(spread * _lane_sel(be), axis=1, keepdims=True)


def _pack_rows(c, be):
    r = be // 128
    return jax.lax.dot_general(_row_sel(be, r), c * _lane_sel(be),
                               (((0,), (0,)), ((), ())),
                               preferred_element_type=jnp.float32)


def _sc_gather_geom(htab, ptab, row_idx, col_idx, e_out):
    n, hd = htab.shape
    pd = ptab.shape[1]
    e = row_idx.shape[0]
    chunks = e // _CHUNK
    iters = -(-chunks // _NW)
    mesh = plsc.VectorSubcoreMesh(core_axis_name="c", subcore_axis_name="s")

    iters2 = -(-iters // 2)

    @functools.partial(
        pl.kernel, mesh=mesh,
        out_type=(jax.ShapeDtypeStruct((e_out, hd), jnp.float32),
                  jax.ShapeDtypeStruct((e_out, hd), jnp.float32),
                  jax.ShapeDtypeStruct((e_out // _CHUNK, _CHUNK), jnp.float32)),
        scratch_types=[
            pltpu.VMEM((2, _CHUNK), jnp.int32),
            pltpu.VMEM((2, _CHUNK), jnp.int32),
            pltpu.VMEM((_CHUNK, hd), jnp.float32),
            pltpu.VMEM((_CHUNK, hd), jnp.float32),
            pltpu.VMEM((_CHUNK, hd), jnp.float32),
            pltpu.VMEM((_CHUNK, hd), jnp.float32),
            pltpu.VMEM((_CHUNK, pd), jnp.float32),
            pltpu.VMEM((_CHUNK, pd), jnp.float32),
            pltpu.VMEM((_CHUNK, pd), jnp.float32),
            pltpu.VMEM((_CHUNK, pd), jnp.float32),
            pltpu.VMEM((_CHUNK,), jnp.float32),
            pltpu.SemaphoreType.DMA,
            pltpu.SemaphoreType.DMA,
            pltpu.SemaphoreType.DMA,
            pltpu.SemaphoreType.DMA,
        ],
        compiler_params=pltpu.CompilerParams(use_tc_tiling_on_sc=False, needs_layout_passes=False),
    )
    def k(h_hbm, p_hbm, r_hbm, c_hbm, gr_hbm, gc_hbm, ss_hbm,
          ri2, ci2, hrb0, hrb1, hcb0, hcb1, prb0, prb1, pcb0, pcb1, ssb,
          sg0, sg1, so0, so1):
        wid = lax.axis_index("s") * _NC + lax.axis_index("c")
        hrb = (hrb0, hrb1)
        hcb = (hcb0, hcb1)
        prb = (prb0, prb1)
        pcb = (pcb0, pcb1)
        sg = (sg0, sg1)
        so = (so0, so1)

        def issue(kk, b):
            ck = wid + kk * _NW

            @pl.when(ck < chunks)
            def _():
                off = pl.multiple_of(ck * _CHUNK, _CHUNK)
                pltpu.sync_copy(r_hbm.at[pl.ds(off, _CHUNK)], ri2.at[b])
                pltpu.sync_copy(c_hbm.at[pl.ds(off, _CHUNK)], ci2.at[b])
                pltpu.async_copy(h_hbm.at[ri2.at[b]], hrb[b], sg[b])
                pltpu.async_copy(h_hbm.at[ci2.at[b]], hcb[b], sg[b])
                pltpu.async_copy(p_hbm.at[ri2.at[b]], prb[b], sg[b])
                pltpu.async_copy(p_hbm.at[ci2.at[b]], pcb[b], sg[b])

        def process(kk, b):
            ck = wid + kk * _NW

            @pl.when(ck < chunks)
            def _():
                off = pl.multiple_of(ck * _CHUNK, _CHUNK)
                pltpu.make_async_copy(h_hbm.at[pl.ds(0, _CHUNK)], hrb[b], sg[b]).wait()
                pltpu.make_async_copy(h_hbm.at[pl.ds(0, _CHUNK)], hcb[b], sg[b]).wait()
                pltpu.make_async_copy(p_hbm.at[pl.ds(0, _CHUNK)], prb[b], sg[b]).wait()
                pltpu.make_async_copy(p_hbm.at[pl.ds(0, _CHUNK)], pcb[b], sg[b]).wait()
                pltpu.async_copy(hrb[b], gr_hbm.at[pl.ds(off, _CHUNK)], so[b])
                pltpu.async_copy(hcb[b], gc_hbm.at[pl.ds(off, _CHUNK)], so[b])

                def geom(g, _):
                    base = pl.multiple_of(g * _L, _L)
                    eidx = base + lax.iota(jnp.int32, _L)
                    acc = jnp.zeros((_L,), jnp.float32)
                    for j in range(3):
                        cj = jnp.full((_L,), j, jnp.int32)
                        dj = (plsc.load_gather(prb[b], [eidx, cj])
                              - plsc.load_gather(pcb[b], [eidx, cj]))
                        acc = acc + dj * dj
                    ssb[pl.ds(base, _L)] = acc
                    return 0

                lax.fori_loop(0, _CHUNK // _L, geom, 0)
                pltpu.sync_copy(ssb, ss_hbm.at[ck])

        def refill(kk, b):
            ck = wid + kk * _NW

            @pl.when(ck < chunks)
            def _():
                pltpu.make_async_copy(h_hbm.at[pl.ds(0, _CHUNK)], hrb[b], so[b]).wait()
                pltpu.make_async_copy(h_hbm.at[pl.ds(0, _CHUNK)], hcb[b], so[b]).wait()
            issue(kk, b)

        issue(0, 0)
        issue(1, 1)

        def body(i, _):
            for b in (0, 1):
                kk = 2 * i + b
                process(kk, b)
                refill(kk + 2, b)
            return 0

        lax.fori_loop(0, iters2, body, 0)

        for b in (0, 1):
            @pl.when(wid + b * _NW < chunks)
            def _(b=b):
                pltpu.make_async_copy(h_hbm.at[pl.ds(0, _CHUNK)], hrb[b], so[b]).wait()
                pltpu.make_async_copy(h_hbm.at[pl.ds(0, _CHUNK)], hcb[b], so[b]).wait()

    return k(htab, ptab, row_idx, col_idx)


def _sc_gather_geom_only(ptab, row_idx, col_idx, e_out):
    n, pd = ptab.shape
    e = row_idx.shape[0]
    chunks = e // _CHUNK
    iters = -(-chunks // _NW)
    mesh = plsc.VectorSubcoreMesh(core_axis_name="c", subcore_axis_name="s")

    iters2 = -(-iters // 2)

    @functools.partial(
        pl.kernel, mesh=mesh,
        out_type=jax.ShapeDtypeStruct((e_out // _CHUNK, _CHUNK), jnp.float32),
        scratch_types=[
            pltpu.VMEM((2, _CHUNK), jnp.int32),
            pltpu.VMEM((2, _CHUNK), jnp.int32),
            pltpu.VMEM((_CHUNK, pd), jnp.float32),
            pltpu.VMEM((_CHUNK, pd), jnp.float32),
            pltpu.VMEM((_CHUNK, pd), jnp.float32),
            pltpu.VMEM((_CHUNK, pd), jnp.float32),
            pltpu.VMEM((_CHUNK,), jnp.float32),
            pltpu.SemaphoreType.DMA,
            pltpu.SemaphoreType.DMA,
        ],
        compiler_params=pltpu.CompilerParams(use_tc_tiling_on_sc=False, needs_layout_passes=False),
    )
    def k(p_hbm, r_hbm, c_hbm, ss_hbm, ri2, ci2, prb0, prb1, pcb0, pcb1, ssb,
          sg0, sg1):
        wid = lax.axis_index("s") * _NC + lax.axis_index("c")
        prb = (prb0, prb1)
        pcb = (pcb0, pcb1)
        sg = (sg0, sg1)

        def issue(kk, b):
            ck = wid + kk * _NW

            @pl.when(ck < chunks)
            def _():
                off = pl.multiple_of(ck * _CHUNK, _CHUNK)
                pltpu.sync_copy(r_hbm.at[pl.ds(off, _CHUNK)], ri2.at[b])
                pltpu.sync_copy(c_hbm.at[pl.ds(off, _CHUNK)], ci2.at[b])
                pltpu.async_copy(p_hbm.at[ri2.at[b]], prb[b], sg[b])
                pltpu.async_copy(p_hbm.at[ci2.at[b]], pcb[b], sg[b])

        def process(kk, b):
            ck = wid + kk * _NW

            @pl.when(ck < chunks)
            def _():
                pltpu.make_async_copy(p_hbm.at[pl.ds(0, _CHUNK)], prb[b], sg[b]).wait()
                pltpu.make_async_copy(p_hbm.at[pl.ds(0, _CHUNK)], pcb[b], sg[b]).wait()

                def geom(g, _):
                    base = pl.multiple_of(g * _L, _L)
                    eidx = base + lax.iota(jnp.int32, _L)
                    acc = jnp.zeros((_L,), jnp.float32)
                    for j in range(3):
                        cj = jnp.full((_L,), j, jnp.int32)
                        dj = (plsc.load_gather(prb[b], [eidx, cj])
                              - plsc.load_gather(pcb[b], [eidx, cj]))
                        acc = acc + dj * dj
                    ssb[pl.ds(base, _L)] = acc
                    return 0

                lax.fori_loop(0, _CHUNK // _L, geom, 0)
                pltpu.sync_copy(ssb, ss_hbm.at[ck])

        issue(0, 0)
        issue(1, 1)

        def body(i, _):
            for b in (0, 1):
                kk = 2 * i + b
                process(kk, b)
                issue(kk + 2, b)
            return 0

        lax.fori_loop(0, iters2, body, 0)

    return k(ptab, row_idx, col_idx)


def _sc_scatter_edge(m_att, cws_packed, row_idx, col_idx, ptab, npad):
    e = row_idx.shape[0]
    hd = m_att.shape[1]
    pd = ptab.shape[1]
    d2 = 8
    chunks = e // _CHUNK
    iters = -(-chunks // _NW)
    tpb = npad // _NS
    z1 = jnp.zeros((tpb, hd), jnp.float32)
    z2 = jnp.zeros((tpb, d2), jnp.float32)
    mesh = plsc.VectorSubcoreMesh(core_axis_name="c", subcore_axis_name="s")

    iters2 = -(-iters // 2)

    @functools.partial(
        pl.kernel, mesh=mesh,
        out_type=(jax.ShapeDtypeStruct((_NC * npad, hd), jnp.float32),
                  jax.ShapeDtypeStruct((_NC * npad, d2), jnp.float32)),
        scratch_types=[
            pltpu.VMEM((2, _CHUNK), jnp.int32),
            pltpu.VMEM((2, _CHUNK), jnp.int32),
            pltpu.VMEM((_CHUNK, hd), jnp.float32),
            pltpu.VMEM((_CHUNK, hd), jnp.float32),
            pltpu.VMEM((_CHUNK, pd), jnp.float32),
            pltpu.VMEM((_CHUNK, pd), jnp.float32),
            pltpu.VMEM((2, _CHUNK), jnp.float32),
            pltpu.VMEM((_CHUNK, d2), jnp.float32),
            pltpu.VMEM_SHARED((npad, hd), jnp.float32),
            pltpu.VMEM_SHARED((npad, d2), jnp.float32),
            pltpu.SemaphoreType.DMA,
            pltpu.SemaphoreType.DMA,
        ],
        compiler_params=pltpu.CompilerParams(use_tc_tiling_on_sc=False, needs_layout_passes=False),
    )
    def k(ma_hbm, cw_hbm, r_hbm, c_hbm, p_hbm, z1_hbm, z2_hbm,
          out1_hbm, out2_hbm, ri2, ci2, pb0, pb1, pcb0, pcb1, cw2b, prod,
          acc1, acc2, sl0, sl1):
        cid = lax.axis_index("c")
        sid = lax.axis_index("s")
        wid = sid * _NC + cid
        pb = (pb0, pb1)
        pcb = (pcb0, pcb1)
        sl = (sl0, sl1)
        zoff = pl.multiple_of(sid * tpb, 8)
        pltpu.sync_copy(z1_hbm, acc1.at[pl.ds(zoff, tpb)])
        pltpu.sync_copy(z2_hbm, acc2.at[pl.ds(zoff, tpb)])
        pltpu.sync_copy(z2_hbm.at[pl.ds(0, _CHUNK)], prod)

        def issue(kk, b):
            ck = wid + kk * _NW

            @pl.when(ck < chunks)
            def _():
                off = pl.multiple_of(ck * _CHUNK, _CHUNK)
                pltpu.sync_copy(r_hbm.at[pl.ds(off, _CHUNK)], ri2.at[b])
                pltpu.sync_copy(c_hbm.at[pl.ds(off, _CHUNK)], ci2.at[b])
                pltpu.async_copy(ma_hbm.at[pl.ds(off, _CHUNK)], pb[b], sl[b])
                pltpu.async_copy(cw_hbm.at[ck], cw2b.at[b], sl[b])
                pltpu.async_copy(p_hbm.at[ci2.at[b]], pcb[b], sl[b])

        def process(kk, b):
            ck = wid + kk * _NW

            @pl.when(ck < chunks)
            def _():
                pltpu.make_async_copy(ma_hbm.at[pl.ds(0, _CHUNK)], pb[b], sl[b]).wait()
                pltpu.make_async_copy(cw_hbm.at[0], cw2b.at[b], sl[b]).wait()
                pltpu.make_async_copy(p_hbm.at[pl.ds(0, _CHUNK)], pcb[b], sl[b]).wait()

                def build(g, _):
                    base = pl.multiple_of(g * _L, _L)
                    eidx = base + lax.iota(jnp.int32, _L)
                    cw16 = cw2b[b, pl.ds(base, _L)]
                    for j in range(3):
                        cj = jnp.full((_L,), j, jnp.int32)
                        v = plsc.load_gather(pcb[b], [eidx, cj]) * cw16
                        plsc.store_scatter(prod, [eidx, cj], v)
                    plsc.store_scatter(prod, [eidx, jnp.full((_L,), 3, jnp.int32)], cw16)
                    plsc.store_scatter(prod, [eidx, jnp.full((_L,), 4, jnp.int32)],
                                       jnp.ones((_L,), jnp.float32))
                    return 0

                lax.fori_loop(0, _CHUNK // _L, build, 0)
                pltpu.sync_copy(pb[b], acc1.at[ri2.at[b]], add=True)
                pltpu.sync_copy(prod, acc2.at[ri2.at[b]], add=True)

        plsc.subcore_barrier()
        issue(0, 0)
        issue(1, 1)

        def body(i, _):
            for b in (0, 1):
                kk = 2 * i + b
                process(kk, b)
                issue(kk + 2, b)
            return 0

        lax.fori_loop(0, iters2, body, 0)
        plsc.subcore_barrier()
        ooff = pl.multiple_of(cid * npad + sid * tpb, 8)
        pltpu.sync_copy(acc1.at[pl.ds(zoff, tpb)], out1_hbm.at[pl.ds(ooff, tpb)])
        pltpu.sync_copy(acc2.at[pl.ds(zoff, tpb)], out2_hbm.at[pl.ds(ooff, tpb)])

    return k(m_att, cws_packed, row_idx, col_idx, ptab, z1, z2)


def _sc_scatter_add(payload, row_idx, npad):
    e = row_idx.shape[0]
    d = payload.shape[1]
    chunks = e // _CHUNK
    iters = -(-chunks // _NW)
    tpb = npad // _NS
    zeros = jnp.zeros((tpb, d), jnp.float32)
    mesh = plsc.VectorSubcoreMesh(core_axis_name="c", subcore_axis_name="s")

    iters2 = -(-iters // 2)

    @functools.partial(
        pl.kernel, mesh=mesh,
        out_type=jax.ShapeDtypeStruct((_NC * npad, d), jnp.float32),
        scratch_types=[
            pltpu.VMEM((2, _CHUNK), jnp.int32),
            pltpu.VMEM((_CHUNK, d), jnp.float32),
            pltpu.VMEM((_CHUNK, d), jnp.float32),
            pltpu.VMEM_SHARED((npad, d), jnp.float32),
            pltpu.SemaphoreType.DMA,
            pltpu.SemaphoreType.DMA,
        ],
        compiler_params=pltpu.CompilerParams(use_tc_tiling_on_sc=False, needs_layout_passes=False),
    )
    def k(p_hbm, r_hbm, z_hbm, out_hbm, ri2, pb0, pb1, acc, sl0, sl1):
        cid = lax.axis_index("c")
        sid = lax.axis_index("s")
        wid = sid * _NC + cid
        pb = (pb0, pb1)
        sl = (sl0, sl1)
        zoff = pl.multiple_of(sid * tpb, 8)
        pltpu.sync_copy(z_hbm, acc.at[pl.ds(zoff, tpb)])

        def issue(kk, b):
            ck = wid + kk * _NW

            @pl.when(ck < chunks)
            def _():
                off = pl.multiple_of(ck * _CHUNK, _CHUNK)
                pltpu.sync_copy(r_hbm.at[pl.ds(off, _CHUNK)], ri2.at[b])
                pltpu.async_copy(p_hbm.at[pl.ds(off, _CHUNK)], pb[b], sl[b])

        def process(kk, b):
            ck = wid + kk * _NW

            @pl.when(ck < chunks)
            def _():
                pltpu.make_async_copy(p_hbm.at[pl.ds(0, _CHUNK)], pb[b], sl[b]).wait()
                pltpu.sync_copy(pb[b], acc.at[ri2.at[b]], add=True)

        plsc.subcore_barrier()
        issue(0, 0)
        issue(1, 1)

        def body(i, _):
            for b in (0, 1):
                kk = 2 * i + b
                process(kk, b)
                issue(kk + 2, b)
            return 0

        lax.fori_loop(0, iters2, body, 0)
        plsc.subcore_barrier()
        ooff = pl.multiple_of(cid * npad + sid * tpb, 8)
        pltpu.sync_copy(acc.at[pl.ds(zoff, tpb)], out_hbm.at[pl.ds(ooff, tpb)])

    return k(payload, row_idx, zeros)


def _full(shape):
    return pl.BlockSpec(shape, lambda i: (0, 0))


def _edge_mlp(hr, hc, ss, ea, wa, wb, wdist, wea, eb1, elg, elb, ew2, eb2,
              awr, ab, cw1, cb1, cw2r):
    e, hd = hr.shape
    be = 2048
    ed = ea.shape[1]

    def body(hr_ref, hc_ref, ss_ref, ea_ref, wa_ref, wb_ref, wd_ref, we_ref,
             eb1_ref, elg_ref, elb_ref, ew2_ref, eb2_ref, awr_ref, ab_ref,
             cw1_ref, cb1_ref, cw2r_ref, ma_ref, cw_ref):
        hr_b = hr_ref[...]
        hc_b = hc_ref[...]
        sel_s = _row_sel(be, be // 128)
        sel_m = _lane_sel(be)
        spread = jax.lax.dot_general(sel_s, ss_ref[...],
                                     (((1,), (0,)), ((), ())),
                                     preferred_element_type=jnp.float32)
        ss_col = jnp.sum(spread * sel_m, axis=1, keepdims=True)
        dist = jnp.sqrt(jnp.maximum(ss_col, 1e-10))
        pre = (_bdot(hr_b, wa_ref[...]) + _bdot(hc_b, wb_ref[...])
               + _bdot(ea_ref[...], we_ref[...])
               + dist * wd_ref[...] + eb1_ref[...])
        x = _layernorm(_silu(pre), elg_ref[...], elb_ref[...])
        m = _silu(_bdot(x, ew2_ref[...]) + eb2_ref[...])
        att = jax.nn.sigmoid(jnp.sum(m * awr_ref[...], axis=1, keepdims=True)
                             + ab_ref[...])
        ma_ref[...] = m * att
        cwv = _silu(_bdot(m, cw1_ref[...]) + cb1_ref[...])
        cws = jnp.sum(cwv * cw2r_ref[...], axis=1, keepdims=True)
        cw_ref[...] = jax.lax.dot_general(sel_s, cws * sel_m,
                                          (((0,), (0,)), ((), ())),
                                          preferred_element_type=jnp.float32)

    return pl.pallas_call(
        body,
        grid=(e // be,),
        in_specs=[
            pl.BlockSpec((be, hd), lambda i: (i, 0)),
            pl.BlockSpec((be, hd), lambda i: (i, 0)),
            pl.BlockSpec((be // 128, 128), lambda i: (i, 0)),
            pl.BlockSpec((be, ed), lambda i: (i, 0)),
            _full(wa.shape), _full(wb.shape), _full(wdist.shape),
            _full(wea.shape), _full(eb1.shape), _full(elg.shape),
            _full(elb.shape), _full(ew2.shape), _full(eb2.shape),
            _full(awr.shape), _full(ab.shape), _full(cw1.shape),
            _full(cb1.shape), _full(cw2r.shape),
        ],
        out_specs=[
            pl.BlockSpec((be, hd), lambda i: (i, 0)),
            pl.BlockSpec((be // 128, 128), lambda i: (i, 0)),
        ],
        out_shape=[
            jax.ShapeDtypeStruct((e, hd), jnp.float32),
            jax.ShapeDtypeStruct((e // 128, 128), jnp.float32),
        ],
    )(hr, hc, ss, ea, wa, wb, wdist, wea, eb1, elg, elb, ew2, eb2,
      awr, ab, cw1, cb1, cw2r)


def _node_mlp(h, ps, qs, pos_pad, nw1a, nw1b, nb1, nlg, nlb, nw2, nb2,
              ng, nb_, fb2r):
    n, hd = h.shape
    np_ = len(ps)
    d2 = qs[0].shape[1]
    pd = pos_pad.shape[1]
    bn = 2000

    def body(h_ref, *refs):
        p_refs = refs[:np_]
        q_refs = refs[np_:2 * np_]
        (pp_ref, w1a_ref, w1b_ref, nb1_ref, nlg_ref, nlb_ref, nw2_ref,
         nb2_ref, ng_ref, nb_ref, fb2_ref, hmid_ref, pn_ref) = refs[2 * np_:]
        hb = h_ref[...]
        agg = sum(r[...] for r in p_refs[1:]) + p_refs[0][...]
        s2 = sum(r[...] for r in q_refs[1:]) + q_refs[0][...]
        scw = s2[:, 3:4]
        deg = s2[:, 4:5]
        spc = jnp.concatenate(
            [s2[:, :3], jnp.zeros((s2.shape[0], pd - 3), jnp.float32)], axis=1)
        pre = (_bdot(hb, w1a_ref[...]) + _bdot(agg, w1b_ref[...])
               + nb1_ref[...])
        y = _layernorm(_silu(pre), nlg_ref[...], nlb_ref[...])
        y = _bdot(y, nw2_ref[...]) + nb2_ref[...]
        hmid = _layernorm(hb + y, ng_ref[...], nb_ref[...])
        hmid_ref[...] = hmid + 0.1 * deg * fb2_ref[...]
        pp = pp_ref[...]
        pn_ref[...] = pp + pp * scw - spc

    return pl.pallas_call(
        body,
        grid=(n // bn,),
        in_specs=[pl.BlockSpec((bn, hd), lambda i: (i, 0))]
        + [pl.BlockSpec((bn, hd), lambda i: (i, 0))] * np_
        + [pl.BlockSpec((bn, d2), lambda i: (i, 0))] * np_
        + [
            pl.BlockSpec((bn, pd), lambda i: (i, 0)),
            _full(nw1a.shape), _full(nw1b.shape), _full(nb1.shape),
            _full(nlg.shape), _full(nlb.shape), _full(nw2.shape),
            _full(nb2.shape), _full(ng.shape), _full(nb_.shape),
            _full(fb2r.shape),
        ],
        out_specs=[
            pl.BlockSpec((bn, hd), lambda i: (i, 0)),
            pl.BlockSpec((bn, pd), lambda i: (i, 0)),
        ],
        out_shape=[
            jax.ShapeDtypeStruct((n, hd), jnp.float32),
            jax.ShapeDtypeStruct((n, pd), jnp.float32),
        ],
    )(h, *ps, *qs, pos_pad, nw1a, nw1b, nb1, nlg, nlb, nw2, nb2,
      ng, nb_, fb2r)


def _edge_dist_stage(ssn, fw1r, fb1r, e):
    hd = fw1r.shape[1]
    be = 4096

    def body(ss_ref, fw1_ref, fb1_ref, out_ref):
        dist = jnp.sqrt(jnp.maximum(_unpack_cols(ss_ref[...], be), 1e-10))
        out_ref[...] = _silu(dist * fw1_ref[...] + fb1_ref[...])

    return pl.pallas_call(
        body,
        grid=(e // be,),
        in_specs=[
            pl.BlockSpec((be // 128, 128), lambda i: (i, 0)),
            _full(fw1r.shape), _full(fb1r.shape),
        ],
        out_specs=pl.BlockSpec((be, hd), lambda i: (i, 0)),
        out_shape=jax.ShapeDtypeStruct((e, hd), jnp.float32),
    )(ssn, fw1r, fb1r)


def _final_stage(hmid, fs, fw2):
    n, hd = hmid.shape
    nf = len(fs)
    bn = 2000

    def body(hm_ref, *refs):
        f_refs = refs[:nf]
        fw2_ref, out_ref = refs[nf:]
        s = sum(r[...] for r in f_refs[1:]) + f_refs[0][...]
        out_ref[...] = hm_ref[...] + 0.1 * _bdot(s, fw2_ref[...])

    return pl.pallas_call(
        body,
        grid=(n // bn,),
        in_specs=[pl.BlockSpec((bn, hd), lambda i: (i, 0))] * (1 + nf)
        + [_full(fw2.shape)],
        out_specs=pl.BlockSpec((bn, hd), lambda i: (i, 0)),
        out_shape=jax.ShapeDtypeStruct((n, hd), jnp.float32),
    )(hmid, *fs, fw2)


def kernel(h, pos, edge_attr, edge_index, ew1, eb1, elg, elb, ew2, eb2,
           nw1, nb1, nlg, nlb, nw2, nb2, ng, nb, cw1, cb1, cw2, aw, ab,
           fw1, fb1, fw2, fb2):
    n, hd = h.shape
    e = edge_index.shape[1]
    row = edge_index[0]
    col = edge_index[1]

    pos_pad = jnp.pad(pos, ((0, 0), (0, 16 - pos.shape[1])))
    r1 = lambda v: v.reshape(1, -1)
    tpb = (-(-n // _NS) + 7) // 8 * 8
    npad = tpb * _NS

    nslab = 2
    es = e // nslab
    es2 = -(-es // 4096) * 4096
    rows = [row[i * es:(i + 1) * es] for i in range(nslab)]
    cols = [col[i * es:(i + 1) * es] for i in range(nslab)]
    eas = [jnp.pad(edge_attr[i * es:(i + 1) * es], ((0, es2 - es), (0, 0)))
           for i in range(nslab)]

    gath = [_sc_gather_geom(h, pos_pad, rows[i], cols[i], es2)
            for i in range(nslab)]
    mlp = [_edge_mlp(
        gath[i][0], gath[i][1], gath[i][2], eas[i],
        ew1[:hd], ew1[hd:2 * hd], ew1[2 * hd:2 * hd + 1], ew1[2 * hd + 1:],
        r1(eb1), r1(elg), r1(elb), ew2, r1(eb2),
        aw.reshape(1, -1), ab.reshape(1, 1), cw1, r1(cb1), cw2.reshape(1, -1))
        for i in range(nslab)]
    scat = [_sc_scatter_edge(mlp[i][0], mlp[i][1], rows[i], cols[i],
                             pos_pad, npad) for i in range(nslab)]
    ps = [s[0][:n] for s in scat] + [s[0][npad:npad + n] for s in scat]
    qs = [s[1][:n] for s in scat] + [s[1][npad:npad + n] for s in scat]

    hmid, pn = _node_mlp(
        h, ps, qs, pos_pad, nw1[:hd], nw1[hd:], r1(nb1), r1(nlg),
        r1(nlb), nw2, r1(nb2), r1(ng), r1(nb), r1(fb2))

    ssn = [_sc_gather_geom_only(pn, rows[i], cols[i], es2)
           for i in range(nslab)]
    s_e = [_edge_dist_stage(ssn[i], fw1, r1(fb1), es2) for i in range(nslab)]
    scat2 = [_sc_scatter_add(s_e[i], rows[i], npad) for i in range(nslab)]
    fs = [s[:n] for s in scat2] + [s[npad:npad + n] for s in scat2]

    h_new = _final_stage(hmid, fs, fw2)
    pos_new = pn[:, :pos.shape[1]]
    return (h_new, pos_new)


# speedup vs baseline: 1.0434x; 1.0434x over previous
import functools

import jax
import jax.numpy as jnp
from jax import lax
from jax.experimental import pallas as pl
from jax.experimental.pallas import tpu as pltpu
from jax.experimental.pallas import tpu_sc as plsc

_NC = 2
_NS = 16
_NW = _NC * _NS
_CHUNK = 128
_L = 16


def _silu(x):
    return x * jax.nn.sigmoid(x)


def _bdot(a, b):
    return jnp.dot(a, b, preferred_element_type=jnp.float32)


def _layernorm(x, g, b):
    mu = jnp.mean(x, axis=-1, keepdims=True)
    var = jnp.mean((x - mu) ** 2, axis=-1, keepdims=True)
    return (x - mu) / jnp.sqrt(var + 1e-5) * g + b


def _row_sel(be, r):
    return (lax.broadcasted_iota(jnp.int32, (be, r), 0) // 128
            == lax.broadcasted_iota(jnp.int32, (be, r), 1)).astype(jnp.float32)


def _lane_sel(be):
    return (lax.broadcasted_iota(jnp.int32, (be, 128), 0) % 128
            == lax.broadcasted_iota(jnp.int32, (be, 128), 1)).astype(jnp.float32)


def _unpack_cols(p, be):
    r = be // 128
    spread = jax.lax.dot_general(_row_sel(be, r), p, (((1,), (0,)), ((), ())),
                                 preferred_element_type=jnp.float32)
    return jnp.sum(spread * _lane_sel(be), axis=1, keepdims=True)


def _pack_rows(c, be):
    r = be // 128
    return jax.lax.dot_general(_row_sel(be, r), c * _lane_sel(be),
                               (((0,), (0,)), ((), ())),
                               preferred_element_type=jnp.float32)


def _sc_gather_geom(htab, ptab, row_idx, col_idx, e_out):
    n, hd = htab.shape
    pd = ptab.shape[1]
    e = row_idx.shape[0]
    chunks = e // _CHUNK
    iters = -(-chunks // _NW)
    mesh = plsc.VectorSubcoreMesh(core_axis_name="c", subcore_axis_name="s")

    iters2 = -(-iters // 2)

    @functools.partial(
        pl.kernel, mesh=mesh,
        out_type=(jax.ShapeDtypeStruct((e_out, hd), jnp.float32),
                  jax.ShapeDtypeStruct((e_out, hd), jnp.float32),
                  jax.ShapeDtypeStruct((e_out // _CHUNK, _CHUNK), jnp.float32)),
        scratch_types=[
            pltpu.VMEM((4, _CHUNK), jnp.int32),
            pltpu.VMEM((4, _CHUNK), jnp.int32),
            pltpu.VMEM((_CHUNK, hd), jnp.float32),
            pltpu.VMEM((_CHUNK, hd), jnp.float32),
            pltpu.VMEM((_CHUNK, hd), jnp.float32),
            pltpu.VMEM((_CHUNK, hd), jnp.float32),
            pltpu.VMEM((_CHUNK, pd), jnp.float32),
            pltpu.VMEM((_CHUNK, pd), jnp.float32),
            pltpu.VMEM((_CHUNK, pd), jnp.float32),
            pltpu.VMEM((_CHUNK, pd), jnp.float32),
            pltpu.VMEM((_CHUNK,), jnp.float32),
            pltpu.SemaphoreType.DMA,
            pltpu.SemaphoreType.DMA,
            pltpu.SemaphoreType.DMA,
            pltpu.SemaphoreType.DMA,
            pltpu.SemaphoreType.DMA,
            pltpu.SemaphoreType.DMA,
            pltpu.SemaphoreType.DMA,
            pltpu.SemaphoreType.DMA,
        ],
        compiler_params=pltpu.CompilerParams(use_tc_tiling_on_sc=False, needs_layout_passes=False),
    )
    def k(h_hbm, p_hbm, r_hbm, c_hbm, gr_hbm, gc_hbm, ss_hbm,
          ri2, ci2, hrb0, hrb1, hcb0, hcb1, prb0, prb1, pcb0, pcb1, ssb,
          sg0, sg1, so0, so1, si0, si1, si2, si3):
        wid = lax.axis_index("s") * _NC + lax.axis_index("c")
        hrb = (hrb0, hrb1)
        hcb = (hcb0, hcb1)
        prb = (prb0, prb1)
        pcb = (pcb0, pcb1)
        sg = (sg0, sg1)
        so = (so0, so1)
        si = (si0, si1, si2, si3)

        def idx_issue(kk, b4):
            ck = wid + kk * _NW

            @pl.when(ck < chunks)
            def _():
                off = pl.multiple_of(ck * _CHUNK, _CHUNK)
                pltpu.async_copy(r_hbm.at[pl.ds(off, _CHUNK)], ri2.at[b4], si[b4])
                pltpu.async_copy(c_hbm.at[pl.ds(off, _CHUNK)], ci2.at[b4], si[b4])

        def issue(kk, b, b4):
            ck = wid + kk * _NW

            @pl.when(ck < chunks)
            def _():
                pltpu.make_async_copy(r_hbm.at[pl.ds(0, _CHUNK)], ri2.at[b4], si[b4]).wait()
                pltpu.make_async_copy(c_hbm.at[pl.ds(0, _CHUNK)], ci2.at[b4], si[b4]).wait()
                pltpu.async_copy(h_hbm.at[ri2.at[b4]], hrb[b], sg[b])
                pltpu.async_copy(h_hbm.at[ci2.at[b4]], hcb[b], sg[b])
                pltpu.async_copy(p_hbm.at[ri2.at[b4]], prb[b], sg[b])
                pltpu.async_copy(p_hbm.at[ci2.at[b4]], pcb[b], sg[b])

        def process(kk, b):
            ck = wid + kk * _NW

            @pl.when(ck < chunks)
            def _():
                off = pl.multiple_of(ck * _CHUNK, _CHUNK)
                pltpu.make_async_copy(h_hbm.at[pl.ds(0, _CHUNK)], hrb[b], sg[b]).wait()
                pltpu.make_async_copy(h_hbm.at[pl.ds(0, _CHUNK)], hcb[b], sg[b]).wait()
                pltpu.make_async_copy(p_hbm.at[pl.ds(0, _CHUNK)], prb[b], sg[b]).wait()
                pltpu.make_async_copy(p_hbm.at[pl.ds(0, _CHUNK)], pcb[b], sg[b]).wait()
                pltpu.async_copy(hrb[b], gr_hbm.at[pl.ds(off, _CHUNK)], so[b])
                pltpu.async_copy(hcb[b], gc_hbm.at[pl.ds(off, _CHUNK)], so[b])

                def geom(g, _):
                    base = pl.multiple_of(g * _L, _L)
                    eidx = base + lax.iota(jnp.int32, _L)
                    acc = jnp.zeros((_L,), jnp.float32)
                    for j in range(3):
                        cj = jnp.full((_L,), j, jnp.int32)
                        dj = (plsc.load_gather(prb[b], [eidx, cj])
                              - plsc.load_gather(pcb[b], [eidx, cj]))
                        acc = acc + dj * dj
                    ssb[pl.ds(base, _L)] = acc
                    return 0

                lax.fori_loop(0, _CHUNK // _L, geom, 0)
                pltpu.sync_copy(ssb, ss_hbm.at[ck])

        def refill(kk, b, b4):
            ck = wid + kk * _NW

            @pl.when(ck < chunks)
            def _():
                pltpu.make_async_copy(h_hbm.at[pl.ds(0, _CHUNK)], hrb[b], so[b]).wait()
                pltpu.make_async_copy(h_hbm.at[pl.ds(0, _CHUNK)], hcb[b], so[b]).wait()
            issue(kk, b, b4)

        for kk0 in range(4):
            idx_issue(kk0, kk0)
        issue(0, 0, 0)
        issue(1, 1, 1)

        iters4 = -(-iters // 4)

        def body(i, _):
            for b4 in (0, 1, 2, 3):
                kk = 4 * i + b4
                b = b4 % 2
                process(kk, b)
                refill(kk + 2, b, (b4 + 2) % 4)
                idx_issue(kk + 4, b4)
            return 0

        lax.fori_loop(0, iters4, body, 0)

        for b in (0, 1):
            @pl.when(wid + b * _NW < chunks)
            def _(b=b):
                pltpu.make_async_copy(h_hbm.at[pl.ds(0, _CHUNK)], hrb[b], so[b]).wait()
                pltpu.make_async_copy(h_hbm.at[pl.ds(0, _CHUNK)], hcb[b], so[b]).wait()

    return k(htab, ptab, row_idx, col_idx)


def _sc_gather_geom_only(ptab, row_idx, col_idx, e_out):
    n, pd = ptab.shape
    e = row_idx.shape[0]
    chunks = e // _CHUNK
    iters = -(-chunks // _NW)
    mesh = plsc.VectorSubcoreMesh(core_axis_name="c", subcore_axis_name="s")

    iters2 = -(-iters // 2)

    @functools.partial(
        pl.kernel, mesh=mesh,
        out_type=jax.ShapeDtypeStruct((e_out // _CHUNK, _CHUNK), jnp.float32),
        scratch_types=[
            pltpu.VMEM((2, _CHUNK), jnp.int32),
            pltpu.VMEM((2, _CHUNK), jnp.int32),
            pltpu.VMEM((_CHUNK, pd), jnp.float32),
            pltpu.VMEM((_CHUNK, pd), jnp.float32),
            pltpu.VMEM((_CHUNK, pd), jnp.float32),
            pltpu.VMEM((_CHUNK, pd), jnp.float32),
            pltpu.VMEM((_CHUNK,), jnp.float32),
            pltpu.SemaphoreType.DMA,
            pltpu.SemaphoreType.DMA,
        ],
        compiler_params=pltpu.CompilerParams(use_tc_tiling_on_sc=False, needs_layout_passes=False),
    )
    def k(p_hbm, r_hbm, c_hbm, ss_hbm, ri2, ci2, prb0, prb1, pcb0, pcb1, ssb,
          sg0, sg1):
        wid = lax.axis_index("s") * _NC + lax.axis_index("c")
        prb = (prb0, prb1)
        pcb = (pcb0, pcb1)
        sg = (sg0, sg1)

        def issue(kk, b):
            ck = wid + kk * _NW

            @pl.when(ck < chunks)
            def _():
                off = pl.multiple_of(ck * _CHUNK, _CHUNK)
                pltpu.sync_copy(r_hbm.at[pl.ds(off, _CHUNK)], ri2.at[b])
                pltpu.sync_copy(c_hbm.at[pl.ds(off, _CHUNK)], ci2.at[b])
                pltpu.async_copy(p_hbm.at[ri2.at[b]], prb[b], sg[b])
                pltpu.async_copy(p_hbm.at[ci2.at[b]], pcb[b], sg[b])

        def process(kk, b):
            ck = wid + kk * _NW

            @pl.when(ck < chunks)
            def _():
                pltpu.make_async_copy(p_hbm.at[pl.ds(0, _CHUNK)], prb[b], sg[b]).wait()
                pltpu.make_async_copy(p_hbm.at[pl.ds(0, _CHUNK)], pcb[b], sg[b]).wait()

                def geom(g, _):
                    base = pl.multiple_of(g * _L, _L)
                    eidx = base + lax.iota(jnp.int32, _L)
                    acc = jnp.zeros((_L,), jnp.float32)
                    for j in range(3):
                        cj = jnp.full((_L,), j, jnp.int32)
                        dj = (plsc.load_gather(prb[b], [eidx, cj])
                              - plsc.load_gather(pcb[b], [eidx, cj]))
                        acc = acc + dj * dj
                    ssb[pl.ds(base, _L)] = acc
                    return 0

                lax.fori_loop(0, _CHUNK // _L, geom, 0)
                pltpu.sync_copy(ssb, ss_hbm.at[ck])

        issue(0, 0)
        issue(1, 1)

        def body(i, _):
            for b in (0, 1):
                kk = 2 * i + b
                process(kk, b)
                issue(kk + 2, b)
            return 0

        lax.fori_loop(0, iters2, body, 0)

    return k(ptab, row_idx, col_idx)


def _sc_scatter_edge(m_att, cws_packed, row_idx, col_idx, ptab, npad):
    e = row_idx.shape[0]
    hd = m_att.shape[1]
    pd = ptab.shape[1]
    d2 = 8
    chunks = e // _CHUNK
    iters = -(-chunks // _NW)
    tpb = npad // _NS
    z1 = jnp.zeros((tpb, hd), jnp.float32)
    z2 = jnp.zeros((tpb, d2), jnp.float32)
    mesh = plsc.VectorSubcoreMesh(core_axis_name="c", subcore_axis_name="s")

    iters2 = -(-iters // 2)

    @functools.partial(
        pl.kernel, mesh=mesh,
        out_type=(jax.ShapeDtypeStruct((_NC * npad, hd), jnp.float32),
                  jax.ShapeDtypeStruct((_NC * npad, d2), jnp.float32)),
        scratch_types=[
            pltpu.VMEM((2, _CHUNK), jnp.int32),
            pltpu.VMEM((2, _CHUNK), jnp.int32),
            pltpu.VMEM((_CHUNK, hd), jnp.float32),
            pltpu.VMEM((_CHUNK, hd), jnp.float32),
            pltpu.VMEM((_CHUNK, pd), jnp.float32),
            pltpu.VMEM((_CHUNK, pd), jnp.float32),
            pltpu.VMEM((2, _CHUNK), jnp.float32),
            pltpu.VMEM((_CHUNK, d2), jnp.float32),
            pltpu.VMEM_SHARED((npad, hd), jnp.float32),
            pltpu.VMEM_SHARED((npad, d2), jnp.float32),
            pltpu.SemaphoreType.DMA,
            pltpu.SemaphoreType.DMA,
        ],
        compiler_params=pltpu.CompilerParams(use_tc_tiling_on_sc=False, needs_layout_passes=False),
    )
    def k(ma_hbm, cw_hbm, r_hbm, c_hbm, p_hbm, z1_hbm, z2_hbm,
          out1_hbm, out2_hbm, ri2, ci2, pb0, pb1, pcb0, pcb1, cw2b, prod,
          acc1, acc2, sl0, sl1):
        cid = lax.axis_index("c")
        sid = lax.axis_index("s")
        wid = sid * _NC + cid
        pb = (pb0, pb1)
        pcb = (pcb0, pcb1)
        sl = (sl0, sl1)
        zoff = pl.multiple_of(sid * tpb, 8)
        pltpu.sync_copy(z1_hbm, acc1.at[pl.ds(zoff, tpb)])
        pltpu.sync_copy(z2_hbm, acc2.at[pl.ds(zoff, tpb)])
        pltpu.sync_copy(z2_hbm.at[pl.ds(0, _CHUNK)], prod)

        def issue(kk, b):
            ck = wid + kk * _NW

            @pl.when(ck < chunks)
            def _():
                off = pl.multiple_of(ck * _CHUNK, _CHUNK)
                pltpu.sync_copy(r_hbm.at[pl.ds(off, _CHUNK)], ri2.at[b])
                pltpu.sync_copy(c_hbm.at[pl.ds(off, _CHUNK)], ci2.at[b])
                pltpu.async_copy(ma_hbm.at[pl.ds(off, _CHUNK)], pb[b], sl[b])
                pltpu.async_copy(cw_hbm.at[ck], cw2b.at[b], sl[b])
                pltpu.async_copy(p_hbm.at[ci2.at[b]], pcb[b], sl[b])

        def process(kk, b):
            ck = wid + kk * _NW

            @pl.when(ck < chunks)
            def _():
                pltpu.make_async_copy(ma_hbm.at[pl.ds(0, _CHUNK)], pb[b], sl[b]).wait()
                pltpu.make_async_copy(cw_hbm.at[0], cw2b.at[b], sl[b]).wait()
                pltpu.make_async_copy(p_hbm.at[pl.ds(0, _CHUNK)], pcb[b], sl[b]).wait()

                def build(g, _):
                    base = pl.multiple_of(g * _L, _L)
                    eidx = base + lax.iota(jnp.int32, _L)
                    cw16 = cw2b[b, pl.ds(base, _L)]
                    for j in range(3):
                        cj = jnp.full((_L,), j, jnp.int32)
                        v = plsc.load_gather(pcb[b], [eidx, cj]) * cw16
                        plsc.store_scatter(prod, [eidx, cj], v)
                    plsc.store_scatter(prod, [eidx, jnp.full((_L,), 3, jnp.int32)], cw16)
                    plsc.store_scatter(prod, [eidx, jnp.full((_L,), 4, jnp.int32)],
                                       jnp.ones((_L,), jnp.float32))
                    return 0

                lax.fori_loop(0, _CHUNK // _L, build, 0)
                pltpu.sync_copy(pb[b], acc1.at[ri2.at[b]], add=True)
                pltpu.sync_copy(prod, acc2.at[ri2.at[b]], add=True)

        plsc.subcore_barrier()
        issue(0, 0)
        issue(1, 1)

        def body(i, _):
            for b in (0, 1):
                kk = 2 * i + b
                process(kk, b)
                issue(kk + 2, b)
            return 0

        lax.fori_loop(0, iters2, body, 0)
        plsc.subcore_barrier()
        ooff = pl.multiple_of(cid * npad + sid * tpb, 8)
        pltpu.sync_copy(acc1.at[pl.ds(zoff, tpb)], out1_hbm.at[pl.ds(ooff, tpb)])
        pltpu.sync_copy(acc2.at[pl.ds(zoff, tpb)], out2_hbm.at[pl.ds(ooff, tpb)])

    return k(m_att, cws_packed, row_idx, col_idx, ptab, z1, z2)


def _sc_scatter_add(payload, row_idx, npad):
    e = row_idx.shape[0]
    d = payload.shape[1]
    chunks = e // _CHUNK
    iters = -(-chunks // _NW)
    tpb = npad // _NS
    zeros = jnp.zeros((tpb, d), jnp.float32)
    mesh = plsc.VectorSubcoreMesh(core_axis_name="c", subcore_axis_name="s")

    iters2 = -(-iters // 2)

    @functools.partial(
        pl.kernel, mesh=mesh,
        out_type=jax.ShapeDtypeStruct((_NC * npad, d), jnp.float32),
        scratch_types=[
            pltpu.VMEM((2, _CHUNK), jnp.int32),
            pltpu.VMEM((_CHUNK, d), jnp.float32),
            pltpu.VMEM((_CHUNK, d), jnp.float32),
            pltpu.VMEM_SHARED((npad, d), jnp.float32),
            pltpu.SemaphoreType.DMA,
            pltpu.SemaphoreType.DMA,
        ],
        compiler_params=pltpu.CompilerParams(use_tc_tiling_on_sc=False, needs_layout_passes=False),
    )
    def k(p_hbm, r_hbm, z_hbm, out_hbm, ri2, pb0, pb1, acc, sl0, sl1):
        cid = lax.axis_index("c")
        sid = lax.axis_index("s")
        wid = sid * _NC + cid
        pb = (pb0, pb1)
        sl = (sl0, sl1)
        zoff = pl.multiple_of(sid * tpb, 8)
        pltpu.sync_copy(z_hbm, acc.at[pl.ds(zoff, tpb)])

        def issue(kk, b):
            ck = wid + kk * _NW

            @pl.when(ck < chunks)
            def _():
                off = pl.multiple_of(ck * _CHUNK, _CHUNK)
                pltpu.sync_copy(r_hbm.at[pl.ds(off, _CHUNK)], ri2.at[b])
                pltpu.async_copy(p_hbm.at[pl.ds(off, _CHUNK)], pb[b], sl[b])

        def process(kk, b):
            ck = wid + kk * _NW

            @pl.when(ck < chunks)
            def _():
                pltpu.make_async_copy(p_hbm.at[pl.ds(0, _CHUNK)], pb[b], sl[b]).wait()
                pltpu.sync_copy(pb[b], acc.at[ri2.at[b]], add=True)

        plsc.subcore_barrier()
        issue(0, 0)
        issue(1, 1)

        def body(i, _):
            for b in (0, 1):
                kk = 2 * i + b
                process(kk, b)
                issue(kk + 2, b)
            return 0

        lax.fori_loop(0, iters2, body, 0)
        plsc.subcore_barrier()
        ooff = pl.multiple_of(cid * npad + sid * tpb, 8)
        pltpu.sync_copy(acc.at[pl.ds(zoff, tpb)], out_hbm.at[pl.ds(ooff, tpb)])

    return k(payload, row_idx, zeros)


def _full(shape):
    return pl.BlockSpec(shape, lambda i: (0, 0))


def _edge_mlp(hr, hc, ss, ea, wa, wb, wdist, wea, eb1, elg, elb, ew2, eb2,
              awr, ab, cw1, cb1, cw2r):
    e, hd = hr.shape
    be = 4096
    ed = ea.shape[1]

    def body(hr_ref, hc_ref, ss_ref, ea_ref, wa_ref, wb_ref, wd_ref, we_ref,
             eb1_ref, elg_ref, elb_ref, ew2_ref, eb2_ref, awr_ref, ab_ref,
             cw1_ref, cb1_ref, cw2r_ref, ma_ref, cw_ref):
        hr_b = hr_ref[...]
        hc_b = hc_ref[...]
        sel_s = _row_sel(be, be // 128)
        sel_m = _lane_sel(be)
        spread = jax.lax.dot_general(sel_s, ss_ref[...],
                                     (((1,), (0,)), ((), ())),
                                     preferred_element_type=jnp.float32)
        ss_col = jnp.sum(spread * sel_m, axis=1, keepdims=True)
        dist = jnp.sqrt(jnp.maximum(ss_col, 1e-10))
        pre = (_bdot(hr_b, wa_ref[...]) + _bdot(hc_b, wb_ref[...])
               + _bdot(ea_ref[...], we_ref[...])
               + dist * wd_ref[...] + eb1_ref[...])
        x = _layernorm(_silu(pre), elg_ref[...], elb_ref[...])
        m = _silu(_bdot(x, ew2_ref[...]) + eb2_ref[...])
        att = jax.nn.sigmoid(jnp.sum(m * awr_ref[...], axis=1, keepdims=True)
                             + ab_ref[...])
        ma_ref[...] = m * att
        cwv = _silu(_bdot(m, cw1_ref[...]) + cb1_ref[...])
        cws = jnp.sum(cwv * cw2r_ref[...], axis=1, keepdims=True)
        cw_ref[...] = jax.lax.dot_general(sel_s, cws * sel_m,
                                          (((0,), (0,)), ((), ())),
                                          preferred_element_type=jnp.float32)

    return pl.pallas_call(
        body,
        grid=(e // be,),
        in_specs=[
            pl.BlockSpec((be, hd), lambda i: (i, 0)),
            pl.BlockSpec((be, hd), lambda i: (i, 0)),
            pl.BlockSpec((be // 128, 128), lambda i: (i, 0)),
            pl.BlockSpec((be, ed), lambda i: (i, 0)),
            _full(wa.shape), _full(wb.shape), _full(wdist.shape),
            _full(wea.shape), _full(eb1.shape), _full(elg.shape),
            _full(elb.shape), _full(ew2.shape), _full(eb2.shape),
            _full(awr.shape), _full(ab.shape), _full(cw1.shape),
            _full(cb1.shape), _full(cw2r.shape),
        ],
        out_specs=[
            pl.BlockSpec((be, hd), lambda i: (i, 0)),
            pl.BlockSpec((be // 128, 128), lambda i: (i, 0)),
        ],
        out_shape=[
            jax.ShapeDtypeStruct((e, hd), jnp.float32),
            jax.ShapeDtypeStruct((e // 128, 128), jnp.float32),
        ],
    )(hr, hc, ss, ea, wa, wb, wdist, wea, eb1, elg, elb, ew2, eb2,
      awr, ab, cw1, cb1, cw2r)


def _node_mlp(h, ps, qs, pos_pad, nw1a, nw1b, nb1, nlg, nlb, nw2, nb2,
              ng, nb_, fb2r):
    n, hd = h.shape
    np_ = len(ps)
    d2 = qs[0].shape[1]
    pd = pos_pad.shape[1]
    bn = 2000

    def body(h_ref, *refs):
        p_refs = refs[:np_]
        q_refs = refs[np_:2 * np_]
        (pp_ref, w1a_ref, w1b_ref, nb1_ref, nlg_ref, nlb_ref, nw2_ref,
         nb2_ref, ng_ref, nb_ref, fb2_ref, hmid_ref, pn_ref) = refs[2 * np_:]
        hb = h_ref[...]
        agg = sum(r[...] for r in p_refs[1:]) + p_refs[0][...]
        s2 = sum(r[...] for r in q_refs[1:]) + q_refs[0][...]
        scw = s2[:, 3:4]
        deg = s2[:, 4:5]
        spc = jnp.concatenate(
            [s2[:, :3], jnp.zeros((s2.shape[0], pd - 3), jnp.float32)], axis=1)
        pre = (_bdot(hb, w1a_ref[...]) + _bdot(agg, w1b_ref[...])
               + nb1_ref[...])
        y = _layernorm(_silu(pre), nlg_ref[...], nlb_ref[...])
        y = _bdot(y, nw2_ref[...]) + nb2_ref[...]
        hmid = _layernorm(hb + y, ng_ref[...], nb_ref[...])
        hmid_ref[...] = hmid + 0.1 * deg * fb2_ref[...]
        pp = pp_ref[...]
        pn_ref[...] = pp + pp * scw - spc

    return pl.pallas_call(
        body,
        grid=(n // bn,),
        in_specs=[pl.BlockSpec((bn, hd), lambda i: (i, 0))]
        + [pl.BlockSpec((bn, hd), lambda i: (i, 0))] * np_
        + [pl.BlockSpec((bn, d2), lambda i: (i, 0))] * np_
        + [
            pl.BlockSpec((bn, pd), lambda i: (i, 0)),
            _full(nw1a.shape), _full(nw1b.shape), _full(nb1.shape),
            _full(nlg.shape), _full(nlb.shape), _full(nw2.shape),
            _full(nb2.shape), _full(ng.shape), _full(nb_.shape),
            _full(fb2r.shape),
        ],
        out_specs=[
            pl.BlockSpec((bn, hd), lambda i: (i, 0)),
            pl.BlockSpec((bn, pd), lambda i: (i, 0)),
        ],
        out_shape=[
            jax.ShapeDtypeStruct((n, hd), jnp.float32),
            jax.ShapeDtypeStruct((n, pd), jnp.float32),
        ],
    )(h, *ps, *qs, pos_pad, nw1a, nw1b, nb1, nlg, nlb, nw2, nb2,
      ng, nb_, fb2r)


def _edge_dist_stage(ssn, fw1r, fb1r, e):
    hd = fw1r.shape[1]
    be = 4096

    def body(ss_ref, fw1_ref, fb1_ref, out_ref):
        dist = jnp.sqrt(jnp.maximum(_unpack_cols(ss_ref[...], be), 1e-10))
        out_ref[...] = _silu(dist * fw1_ref[...] + fb1_ref[...])

    return pl.pallas_call(
        body,
        grid=(e // be,),
        in_specs=[
            pl.BlockSpec((be // 128, 128), lambda i: (i, 0)),
            _full(fw1r.shape), _full(fb1r.shape),
        ],
        out_specs=pl.BlockSpec((be, hd), lambda i: (i, 0)),
        out_shape=jax.ShapeDtypeStruct((e, hd), jnp.float32),
    )(ssn, fw1r, fb1r)


def _final_stage(hmid, fs, fw2):
    n, hd = hmid.shape
    nf = len(fs)
    bn = 2000

    def body(hm_ref, *refs):
        f_refs = refs[:nf]
        fw2_ref, out_ref = refs[nf:]
        s = sum(r[...] for r in f_refs[1:]) + f_refs[0][...]
        out_ref[...] = hm_ref[...] + 0.1 * _bdot(s, fw2_ref[...])

    return pl.pallas_call(
        body,
        grid=(n // bn,),
        in_specs=[pl.BlockSpec((bn, hd), lambda i: (i, 0))] * (1 + nf)
        + [_full(fw2.shape)],
        out_specs=pl.BlockSpec((bn, hd), lambda i: (i, 0)),
        out_shape=jax.ShapeDtypeStruct((n, hd), jnp.float32),
    )(hmid, *fs, fw2)


def kernel(h, pos, edge_attr, edge_index, ew1, eb1, elg, elb, ew2, eb2,
           nw1, nb1, nlg, nlb, nw2, nb2, ng, nb, cw1, cb1, cw2, aw, ab,
           fw1, fb1, fw2, fb2):
    n, hd = h.shape
    e = edge_index.shape[1]
    row = edge_index[0]
    col = edge_index[1]

    pos_pad = jnp.pad(pos, ((0, 0), (0, 16 - pos.shape[1])))
    r1 = lambda v: v.reshape(1, -1)
    tpb = (-(-n // _NS) + 7) // 8 * 8
    npad = tpb * _NS

    nslab = 2
    es = e // nslab
    es2 = -(-es // 4096) * 4096
    rows = [row[i * es:(i + 1) * es] for i in range(nslab)]
    cols = [col[i * es:(i + 1) * es] for i in range(nslab)]
    eas = [jnp.pad(edge_attr[i * es:(i + 1) * es], ((0, es2 - es), (0, 0)))
           for i in range(nslab)]

    gath = [_sc_gather_geom(h, pos_pad, rows[i], cols[i], es2)
            for i in range(nslab)]
    mlp = [_edge_mlp(
        gath[i][0], gath[i][1], gath[i][2], eas[i],
        ew1[:hd], ew1[hd:2 * hd], ew1[2 * hd:2 * hd + 1], ew1[2 * hd + 1:],
        r1(eb1), r1(elg), r1(elb), ew2, r1(eb2),
        aw.reshape(1, -1), ab.reshape(1, 1), cw1, r1(cb1), cw2.reshape(1, -1))
        for i in range(nslab)]
    scat = [_sc_scatter_edge(mlp[i][0], mlp[i][1], rows[i], cols[i],
                             pos_pad, npad) for i in range(nslab)]
    ps = [s[0][:n] for s in scat] + [s[0][npad:npad + n] for s in scat]
    qs = [s[1][:n] for s in scat] + [s[1][npad:npad + n] for s in scat]

    hmid, pn = _node_mlp(
        h, ps, qs, pos_pad, nw1[:hd], nw1[hd:], r1(nb1), r1(nlg),
        r1(nlb), nw2, r1(nb2), r1(ng), r1(nb), r1(fb2))

    ssn = [_sc_gather_geom_only(pn, rows[i], cols[i], es2)
           for i in range(nslab)]
    s_e = [_edge_dist_stage(ssn[i], fw1, r1(fb1), es2) for i in range(nslab)]
    scat2 = [_sc_scatter_add(s_e[i], rows[i], npad) for i in range(nslab)]
    fs = [s[:n] for s in scat2] + [s[npad:npad + n] for s in scat2]

    h_new = _final_stage(hmid, fs, fw2)
    pos_new = pn[:, :pos.shape[1]]
    return (h_new, pos_new)


# speedup vs baseline: 1.0518x; 1.0080x over previous
import functools

import jax
import jax.numpy as jnp
from jax import lax
from jax.experimental import pallas as pl
from jax.experimental.pallas import tpu as pltpu
from jax.experimental.pallas import tpu_sc as plsc

_NC = 2
_NS = 16
_NW = _NC * _NS
_CHUNK = 128
_L = 16


def _silu(x):
    return x * jax.nn.sigmoid(x)


def _bdot(a, b):
    return jnp.dot(a, b, preferred_element_type=jnp.float32)


def _layernorm(x, g, b):
    mu = jnp.mean(x, axis=-1, keepdims=True)
    var = jnp.mean((x - mu) ** 2, axis=-1, keepdims=True)
    return (x - mu) / jnp.sqrt(var + 1e-5) * g + b


def _row_sel(be, r):
    return (lax.broadcasted_iota(jnp.int32, (be, r), 0) // 128
            == lax.broadcasted_iota(jnp.int32, (be, r), 1)).astype(jnp.float32)


def _lane_sel(be):
    return (lax.broadcasted_iota(jnp.int32, (be, 128), 0) % 128
            == lax.broadcasted_iota(jnp.int32, (be, 128), 1)).astype(jnp.float32)


def _unpack_cols(p, be):
    r = be // 128
    spread = jax.lax.dot_general(_row_sel(be, r), p, (((1,), (0,)), ((), ())),
                                 preferred_element_type=jnp.float32)
    return jnp.sum(spread * _lane_sel(be), axis=1, keepdims=True)


def _pack_rows(c, be):
    r = be // 128
    return jax.lax.dot_general(_row_sel(be, r), c * _lane_sel(be),
                               (((0,), (0,)), ((), ())),
                               preferred_element_type=jnp.float32)


def _sc_gather_geom(htab, ptab, row_idx, col_idx, e_out):
    n, hd = htab.shape
    pd = ptab.shape[1]
    e = row_idx.shape[0]
    chunks = e // _CHUNK
    iters = -(-chunks // _NW)
    mesh = plsc.VectorSubcoreMesh(core_axis_name="c", subcore_axis_name="s")

    iters2 = -(-iters // 2)

    @functools.partial(
        pl.kernel, mesh=mesh,
        out_type=(jax.ShapeDtypeStruct((e_out, hd), jnp.float32),
                  jax.ShapeDtypeStruct((e_out, hd), jnp.float32),
                  jax.ShapeDtypeStruct((e_out // _CHUNK, _CHUNK), jnp.float32)),
        scratch_types=[
            pltpu.VMEM((4, _CHUNK), jnp.int32),
            pltpu.VMEM((4, _CHUNK), jnp.int32),
            pltpu.VMEM((_CHUNK, hd), jnp.float32),
            pltpu.VMEM((_CHUNK, hd), jnp.float32),
            pltpu.VMEM((_CHUNK, hd), jnp.float32),
            pltpu.VMEM((_CHUNK, hd), jnp.float32),
            pltpu.VMEM((_CHUNK, pd), jnp.float32),
            pltpu.VMEM((_CHUNK, pd), jnp.float32),
            pltpu.VMEM((_CHUNK, pd), jnp.float32),
            pltpu.VMEM((_CHUNK, pd), jnp.float32),
            pltpu.VMEM((_CHUNK,), jnp.float32),
            pltpu.SemaphoreType.DMA,
            pltpu.SemaphoreType.DMA,
            pltpu.SemaphoreType.DMA,
            pltpu.SemaphoreType.DMA,
            pltpu.SemaphoreType.DMA,
            pltpu.SemaphoreType.DMA,
            pltpu.SemaphoreType.DMA,
            pltpu.SemaphoreType.DMA,
        ],
        compiler_params=pltpu.CompilerParams(use_tc_tiling_on_sc=False, needs_layout_passes=False),
    )
    def k(h_hbm, p_hbm, r_hbm, c_hbm, gr_hbm, gc_hbm, ss_hbm,
          ri2, ci2, hrb0, hrb1, hcb0, hcb1, prb0, prb1, pcb0, pcb1, ssb,
          sg0, sg1, so0, so1, si0, si1, si2, si3):
        wid = lax.axis_index("s") * _NC + lax.axis_index("c")
        hrb = (hrb0, hrb1)
        hcb = (hcb0, hcb1)
        prb = (prb0, prb1)
        pcb = (pcb0, pcb1)
        sg = (sg0, sg1)
        so = (so0, so1)
        si = (si0, si1, si2, si3)

        def idx_issue(kk, b4):
            ck = wid + kk * _NW

            @pl.when(ck < chunks)
            def _():
                off = pl.multiple_of(ck * _CHUNK, _CHUNK)
                pltpu.async_copy(r_hbm.at[pl.ds(off, _CHUNK)], ri2.at[b4], si[b4])
                pltpu.async_copy(c_hbm.at[pl.ds(off, _CHUNK)], ci2.at[b4], si[b4])

        def issue(kk, b, b4):
            ck = wid + kk * _NW

            @pl.when(ck < chunks)
            def _():
                pltpu.make_async_copy(r_hbm.at[pl.ds(0, _CHUNK)], ri2.at[b4], si[b4]).wait()
                pltpu.make_async_copy(c_hbm.at[pl.ds(0, _CHUNK)], ci2.at[b4], si[b4]).wait()
                pltpu.async_copy(h_hbm.at[ri2.at[b4]], hrb[b], sg[b])
                pltpu.async_copy(h_hbm.at[ci2.at[b4]], hcb[b], sg[b])
                pltpu.async_copy(p_hbm.at[ri2.at[b4]], prb[b], sg[b])
                pltpu.async_copy(p_hbm.at[ci2.at[b4]], pcb[b], sg[b])

        def process(kk, b):
            ck = wid + kk * _NW

            @pl.when(ck < chunks)
            def _():
                off = pl.multiple_of(ck * _CHUNK, _CHUNK)
                pltpu.make_async_copy(h_hbm.at[pl.ds(0, _CHUNK)], hrb[b], sg[b]).wait()
                pltpu.make_async_copy(h_hbm.at[pl.ds(0, _CHUNK)], hcb[b], sg[b]).wait()
                pltpu.make_async_copy(p_hbm.at[pl.ds(0, _CHUNK)], prb[b], sg[b]).wait()
                pltpu.make_async_copy(p_hbm.at[pl.ds(0, _CHUNK)], pcb[b], sg[b]).wait()
                pltpu.async_copy(hrb[b], gr_hbm.at[pl.ds(off, _CHUNK)], so[b])
                pltpu.async_copy(hcb[b], gc_hbm.at[pl.ds(off, _CHUNK)], so[b])

                def geom(g, _):
                    base = pl.multiple_of(g * _L, _L)
                    eidx = base + lax.iota(jnp.int32, _L)
                    acc = jnp.zeros((_L,), jnp.float32)
                    for j in range(3):
                        cj = jnp.full((_L,), j, jnp.int32)
                        dj = (plsc.load_gather(prb[b], [eidx, cj])
                              - plsc.load_gather(pcb[b], [eidx, cj]))
                        acc = acc + dj * dj
                    ssb[pl.ds(base, _L)] = acc
                    return 0

                lax.fori_loop(0, _CHUNK // _L, geom, 0)
                pltpu.sync_copy(ssb, ss_hbm.at[ck])

        def refill(kk, b, b4):
            ck = wid + kk * _NW

            @pl.when(ck < chunks)
            def _():
                pltpu.make_async_copy(h_hbm.at[pl.ds(0, _CHUNK)], hrb[b], so[b]).wait()
                pltpu.make_async_copy(h_hbm.at[pl.ds(0, _CHUNK)], hcb[b], so[b]).wait()
            issue(kk, b, b4)

        for kk0 in range(4):
            idx_issue(kk0, kk0)
        issue(0, 0, 0)
        issue(1, 1, 1)

        iters4 = -(-iters // 4)

        def body(i, _):
            for b4 in (0, 1, 2, 3):
                kk = 4 * i + b4
                b = b4 % 2
                process(kk, b)
                refill(kk + 2, b, (b4 + 2) % 4)
                idx_issue(kk + 4, b4)
            return 0

        lax.fori_loop(0, iters4, body, 0)

        for b in (0, 1):
            @pl.when(wid + b * _NW < chunks)
            def _(b=b):
                pltpu.make_async_copy(h_hbm.at[pl.ds(0, _CHUNK)], hrb[b], so[b]).wait()
                pltpu.make_async_copy(h_hbm.at[pl.ds(0, _CHUNK)], hcb[b], so[b]).wait()

    return k(htab, ptab, row_idx, col_idx)


def _sc_gather_geom_only(ptab, row_idx, col_idx, e_out):
    n, pd = ptab.shape
    e = row_idx.shape[0]
    chunks = e // _CHUNK
    iters = -(-chunks // _NW)
    mesh = plsc.VectorSubcoreMesh(core_axis_name="c", subcore_axis_name="s")

    iters2 = -(-iters // 2)

    @functools.partial(
        pl.kernel, mesh=mesh,
        out_type=jax.ShapeDtypeStruct((e_out // _CHUNK, _CHUNK), jnp.float32),
        scratch_types=[
            pltpu.VMEM((2, _CHUNK), jnp.int32),
            pltpu.VMEM((2, _CHUNK), jnp.int32),
            pltpu.VMEM((_CHUNK, pd), jnp.float32),
            pltpu.VMEM((_CHUNK, pd), jnp.float32),
            pltpu.VMEM((_CHUNK, pd), jnp.float32),
            pltpu.VMEM((_CHUNK, pd), jnp.float32),
            pltpu.VMEM((_CHUNK,), jnp.float32),
            pltpu.SemaphoreType.DMA,
            pltpu.SemaphoreType.DMA,
        ],
        compiler_params=pltpu.CompilerParams(use_tc_tiling_on_sc=False, needs_layout_passes=False),
    )
    def k(p_hbm, r_hbm, c_hbm, ss_hbm, ri2, ci2, prb0, prb1, pcb0, pcb1, ssb,
          sg0, sg1):
        wid = lax.axis_index("s") * _NC + lax.axis_index("c")
        prb = (prb0, prb1)
        pcb = (pcb0, pcb1)
        sg = (sg0, sg1)

        def issue(kk, b):
            ck = wid + kk * _NW

            @pl.when(ck < chunks)
            def _():
                off = pl.multiple_of(ck * _CHUNK, _CHUNK)
                pltpu.sync_copy(r_hbm.at[pl.ds(off, _CHUNK)], ri2.at[b])
                pltpu.sync_copy(c_hbm.at[pl.ds(off, _CHUNK)], ci2.at[b])
                pltpu.async_copy(p_hbm.at[ri2.at[b]], prb[b], sg[b])
                pltpu.async_copy(p_hbm.at[ci2.at[b]], pcb[b], sg[b])

        def process(kk, b):
            ck = wid + kk * _NW

            @pl.when(ck < chunks)
            def _():
                pltpu.make_async_copy(p_hbm.at[pl.ds(0, _CHUNK)], prb[b], sg[b]).wait()
                pltpu.make_async_copy(p_hbm.at[pl.ds(0, _CHUNK)], pcb[b], sg[b]).wait()

                def geom(g, _):
                    base = pl.multiple_of(g * _L, _L)
                    eidx = base + lax.iota(jnp.int32, _L)
                    acc = jnp.zeros((_L,), jnp.float32)
                    for j in range(3):
                        cj = jnp.full((_L,), j, jnp.int32)
                        dj = (plsc.load_gather(prb[b], [eidx, cj])
                              - plsc.load_gather(pcb[b], [eidx, cj]))
                        acc = acc + dj * dj
                    ssb[pl.ds(base, _L)] = acc
                    return 0

                lax.fori_loop(0, _CHUNK // _L, geom, 0)
                pltpu.sync_copy(ssb, ss_hbm.at[ck])

        issue(0, 0)
        issue(1, 1)

        def body(i, _):
            for b in (0, 1):
                kk = 2 * i + b
                process(kk, b)
                issue(kk + 2, b)
            return 0

        lax.fori_loop(0, iters2, body, 0)

    return k(ptab, row_idx, col_idx)


def _sc_scatter_edge(m_att, cws_packed, row_idx, col_idx, ptab, npad):
    e = row_idx.shape[0]
    hd = m_att.shape[1]
    pd = ptab.shape[1]
    d2 = 8
    chunks = e // _CHUNK
    iters = -(-chunks // _NW)
    tpb = npad // _NS
    z1 = jnp.zeros((tpb, hd), jnp.float32)
    z2 = jnp.zeros((tpb, d2), jnp.float32)
    mesh = plsc.VectorSubcoreMesh(core_axis_name="c", subcore_axis_name="s")

    iters2 = -(-iters // 2)

    @functools.partial(
        pl.kernel, mesh=mesh,
        out_type=(jax.ShapeDtypeStruct((_NC * npad, hd), jnp.float32),
                  jax.ShapeDtypeStruct((_NC * npad, d2), jnp.float32)),
        scratch_types=[
            pltpu.VMEM((2, _CHUNK), jnp.int32),
            pltpu.VMEM((2, _CHUNK), jnp.int32),
            pltpu.VMEM((_CHUNK, hd), jnp.float32),
            pltpu.VMEM((_CHUNK, hd), jnp.float32),
            pltpu.VMEM((_CHUNK, pd), jnp.float32),
            pltpu.VMEM((_CHUNK, pd), jnp.float32),
            pltpu.VMEM((2, _CHUNK), jnp.float32),
            pltpu.VMEM((_CHUNK, d2), jnp.float32),
            pltpu.VMEM_SHARED((npad, hd), jnp.float32),
            pltpu.VMEM_SHARED((npad, d2), jnp.float32),
            pltpu.SemaphoreType.DMA,
            pltpu.SemaphoreType.DMA,
        ],
        compiler_params=pltpu.CompilerParams(use_tc_tiling_on_sc=False, needs_layout_passes=False),
    )
    def k(ma_hbm, cw_hbm, r_hbm, c_hbm, p_hbm, z1_hbm, z2_hbm,
          out1_hbm, out2_hbm, ri2, ci2, pb0, pb1, pcb0, pcb1, cw2b, prod,
          acc1, acc2, sl0, sl1):
        cid = lax.axis_index("c")
        sid = lax.axis_index("s")
        wid = sid * _NC + cid
        pb = (pb0, pb1)
        pcb = (pcb0, pcb1)
        sl = (sl0, sl1)
        zoff = pl.multiple_of(sid * tpb, 8)
        pltpu.sync_copy(z1_hbm, acc1.at[pl.ds(zoff, tpb)])
        pltpu.sync_copy(z2_hbm, acc2.at[pl.ds(zoff, tpb)])
        pltpu.sync_copy(z2_hbm.at[pl.ds(0, _CHUNK)], prod)

        def issue(kk, b):
            ck = wid + kk * _NW

            @pl.when(ck < chunks)
            def _():
                off = pl.multiple_of(ck * _CHUNK, _CHUNK)
                pltpu.sync_copy(r_hbm.at[pl.ds(off, _CHUNK)], ri2.at[b])
                pltpu.sync_copy(c_hbm.at[pl.ds(off, _CHUNK)], ci2.at[b])
                pltpu.async_copy(ma_hbm.at[pl.ds(off, _CHUNK)], pb[b], sl[b])
                pltpu.async_copy(cw_hbm.at[ck], cw2b.at[b], sl[b])
                pltpu.async_copy(p_hbm.at[ci2.at[b]], pcb[b], sl[b])

        def process(kk, b):
            ck = wid + kk * _NW

            @pl.when(ck < chunks)
            def _():
                pltpu.make_async_copy(ma_hbm.at[pl.ds(0, _CHUNK)], pb[b], sl[b]).wait()
                pltpu.make_async_copy(cw_hbm.at[0], cw2b.at[b], sl[b]).wait()
                pltpu.make_async_copy(p_hbm.at[pl.ds(0, _CHUNK)], pcb[b], sl[b]).wait()

                def build(g, _):
                    base = pl.multiple_of(g * _L, _L)
                    eidx = base + lax.iota(jnp.int32, _L)
                    cw16 = cw2b[b, pl.ds(base, _L)]
                    for j in range(3):
                        cj = jnp.full((_L,), j, jnp.int32)
                        v = plsc.load_gather(pcb[b], [eidx, cj]) * cw16
                        plsc.store_scatter(prod, [eidx, cj], v)
                    plsc.store_scatter(prod, [eidx, jnp.full((_L,), 3, jnp.int32)], cw16)
                    plsc.store_scatter(prod, [eidx, jnp.full((_L,), 4, jnp.int32)],
                                       jnp.ones((_L,), jnp.float32))
                    return 0

                lax.fori_loop(0, _CHUNK // _L, build, 0)
                pltpu.sync_copy(pb[b], acc1.at[ri2.at[b]], add=True)
                pltpu.sync_copy(prod, acc2.at[ri2.at[b]], add=True)

        plsc.subcore_barrier()
        issue(0, 0)
        issue(1, 1)

        def body(i, _):
            for b in (0, 1):
                kk = 2 * i + b
                process(kk, b)
                issue(kk + 2, b)
            return 0

        lax.fori_loop(0, iters2, body, 0)
        plsc.subcore_barrier()
        ooff = pl.multiple_of(cid * npad + sid * tpb, 8)
        pltpu.sync_copy(acc1.at[pl.ds(zoff, tpb)], out1_hbm.at[pl.ds(ooff, tpb)])
        pltpu.sync_copy(acc2.at[pl.ds(zoff, tpb)], out2_hbm.at[pl.ds(ooff, tpb)])

    return k(m_att, cws_packed, row_idx, col_idx, ptab, z1, z2)


def _sc_scatter_add(payload, row_idx, npad):
    e = row_idx.shape[0]
    d = payload.shape[1]
    chunks = e // _CHUNK
    iters = -(-chunks // _NW)
    tpb = npad // _NS
    zeros = jnp.zeros((tpb, d), jnp.float32)
    mesh = plsc.VectorSubcoreMesh(core_axis_name="c", subcore_axis_name="s")

    iters2 = -(-iters // 2)

    @functools.partial(
        pl.kernel, mesh=mesh,
        out_type=jax.ShapeDtypeStruct((_NC * npad, d), jnp.float32),
        scratch_types=[
            pltpu.VMEM((2, _CHUNK), jnp.int32),
            pltpu.VMEM((_CHUNK, d), jnp.float32),
            pltpu.VMEM((_CHUNK, d), jnp.float32),
            pltpu.VMEM_SHARED((npad, d), jnp.float32),
            pltpu.SemaphoreType.DMA,
            pltpu.SemaphoreType.DMA,
        ],
        compiler_params=pltpu.CompilerParams(use_tc_tiling_on_sc=False, needs_layout_passes=False),
    )
    def k(p_hbm, r_hbm, z_hbm, out_hbm, ri2, pb0, pb1, acc, sl0, sl1):
        cid = lax.axis_index("c")
        sid = lax.axis_index("s")
        wid = sid * _NC + cid
        pb = (pb0, pb1)
        sl = (sl0, sl1)
        zoff = pl.multiple_of(sid * tpb, 8)
        pltpu.sync_copy(z_hbm, acc.at[pl.ds(zoff, tpb)])

        def issue(kk, b):
            ck = wid + kk * _NW

            @pl.when(ck < chunks)
            def _():
                off = pl.multiple_of(ck * _CHUNK, _CHUNK)
                pltpu.sync_copy(r_hbm.at[pl.ds(off, _CHUNK)], ri2.at[b])
                pltpu.async_copy(p_hbm.at[pl.ds(off, _CHUNK)], pb[b], sl[b])

        def process(kk, b):
            ck = wid + kk * _NW

            @pl.when(ck < chunks)
            def _():
                pltpu.make_async_copy(p_hbm.at[pl.ds(0, _CHUNK)], pb[b], sl[b]).wait()
                pltpu.sync_copy(pb[b], acc.at[ri2.at[b]], add=True)

        plsc.subcore_barrier()
        issue(0, 0)
        issue(1, 1)

        def body(i, _):
            for b in (0, 1):
                kk = 2 * i + b
                process(kk, b)
                issue(kk + 2, b)
            return 0

        lax.fori_loop(0, iters2, body, 0)
        plsc.subcore_barrier()
        ooff = pl.multiple_of(cid * npad + sid * tpb, 8)
        pltpu.sync_copy(acc.at[pl.ds(zoff, tpb)], out_hbm.at[pl.ds(ooff, tpb)])

    return k(payload, row_idx, zeros)


def _full(shape):
    return pl.BlockSpec(shape, lambda i: (0, 0))


def _edge_mlp(hr, hc, ss, ea, wa, wb, wdist, wea, eb1, elg, elb, ew2, eb2,
              awr, ab, cw1, cb1, cw2r):
    e, hd = hr.shape
    be = 8192
    ed = ea.shape[1]

    def body(hr_ref, hc_ref, ss_ref, ea_ref, wa_ref, wb_ref, wd_ref, we_ref,
             eb1_ref, elg_ref, elb_ref, ew2_ref, eb2_ref, awr_ref, ab_ref,
             cw1_ref, cb1_ref, cw2r_ref, ma_ref, cw_ref):
        hr_b = hr_ref[...]
        hc_b = hc_ref[...]
        sel_s = _row_sel(be, be // 128)
        sel_m = _lane_sel(be)
        spread = jax.lax.dot_general(sel_s, ss_ref[...],
                                     (((1,), (0,)), ((), ())),
                                     preferred_element_type=jnp.float32)
        ss_col = jnp.sum(spread * sel_m, axis=1, keepdims=True)
        dist = jnp.sqrt(jnp.maximum(ss_col, 1e-10))
        pre = (_bdot(hr_b, wa_ref[...]) + _bdot(hc_b, wb_ref[...])
               + _bdot(ea_ref[...], we_ref[...])
               + dist * wd_ref[...] + eb1_ref[...])
        x = _layernorm(_silu(pre), elg_ref[...], elb_ref[...])
        m = _silu(_bdot(x, ew2_ref[...]) + eb2_ref[...])
        att = jax.nn.sigmoid(jnp.sum(m * awr_ref[...], axis=1, keepdims=True)
                             + ab_ref[...])
        ma_ref[...] = m * att
        cwv = _silu(_bdot(m, cw1_ref[...]) + cb1_ref[...])
        cws = jnp.sum(cwv * cw2r_ref[...], axis=1, keepdims=True)
        cw_ref[...] = jax.lax.dot_general(sel_s, cws * sel_m,
                                          (((0,), (0,)), ((), ())),
                                          preferred_element_type=jnp.float32)

    return pl.pallas_call(
        body,
        grid=(e // be,),
        in_specs=[
            pl.BlockSpec((be, hd), lambda i: (i, 0)),
            pl.BlockSpec((be, hd), lambda i: (i, 0)),
            pl.BlockSpec((be // 128, 128), lambda i: (i, 0)),
            pl.BlockSpec((be, ed), lambda i: (i, 0)),
            _full(wa.shape), _full(wb.shape), _full(wdist.shape),
            _full(wea.shape), _full(eb1.shape), _full(elg.shape),
            _full(elb.shape), _full(ew2.shape), _full(eb2.shape),
            _full(awr.shape), _full(ab.shape), _full(cw1.shape),
            _full(cb1.shape), _full(cw2r.shape),
        ],
        out_specs=[
            pl.BlockSpec((be, hd), lambda i: (i, 0)),
            pl.BlockSpec((be // 128, 128), lambda i: (i, 0)),
        ],
        out_shape=[
            jax.ShapeDtypeStruct((e, hd), jnp.float32),
            jax.ShapeDtypeStruct((e // 128, 128), jnp.float32),
        ],
    )(hr, hc, ss, ea, wa, wb, wdist, wea, eb1, elg, elb, ew2, eb2,
      awr, ab, cw1, cb1, cw2r)


def _node_mlp(h, ps, qs, pos_pad, nw1a, nw1b, nb1, nlg, nlb, nw2, nb2,
              ng, nb_, fb2r):
    n, hd = h.shape
    np_ = len(ps)
    d2 = qs[0].shape[1]
    pd = pos_pad.shape[1]
    bn = 2000

    def body(h_ref, *refs):
        p_refs = refs[:np_]
        q_refs = refs[np_:2 * np_]
        (pp_ref, w1a_ref, w1b_ref, nb1_ref, nlg_ref, nlb_ref, nw2_ref,
         nb2_ref, ng_ref, nb_ref, fb2_ref, hmid_ref, pn_ref) = refs[2 * np_:]
        hb = h_ref[...]
        agg = sum(r[...] for r in p_refs[1:]) + p_refs[0][...]
        s2 = sum(r[...] for r in q_refs[1:]) + q_refs[0][...]
        scw = s2[:, 3:4]
        deg = s2[:, 4:5]
        spc = jnp.concatenate(
            [s2[:, :3], jnp.zeros((s2.shape[0], pd - 3), jnp.float32)], axis=1)
        pre = (_bdot(hb, w1a_ref[...]) + _bdot(agg, w1b_ref[...])
               + nb1_ref[...])
        y = _layernorm(_silu(pre), nlg_ref[...], nlb_ref[...])
        y = _bdot(y, nw2_ref[...]) + nb2_ref[...]
        hmid = _layernorm(hb + y, ng_ref[...], nb_ref[...])
        hmid_ref[...] = hmid + 0.1 * deg * fb2_ref[...]
        pp = pp_ref[...]
        pn_ref[...] = pp + pp * scw - spc

    return pl.pallas_call(
        body,
        grid=(n // bn,),
        in_specs=[pl.BlockSpec((bn, hd), lambda i: (i, 0))]
        + [pl.BlockSpec((bn, hd), lambda i: (i, 0))] * np_
        + [pl.BlockSpec((bn, d2), lambda i: (i, 0))] * np_
        + [
            pl.BlockSpec((bn, pd), lambda i: (i, 0)),
            _full(nw1a.shape), _full(nw1b.shape), _full(nb1.shape),
            _full(nlg.shape), _full(nlb.shape), _full(nw2.shape),
            _full(nb2.shape), _full(ng.shape), _full(nb_.shape),
            _full(fb2r.shape),
        ],
        out_specs=[
            pl.BlockSpec((bn, hd), lambda i: (i, 0)),
            pl.BlockSpec((bn, pd), lambda i: (i, 0)),
        ],
        out_shape=[
            jax.ShapeDtypeStruct((n, hd), jnp.float32),
            jax.ShapeDtypeStruct((n, pd), jnp.float32),
        ],
    )(h, *ps, *qs, pos_pad, nw1a, nw1b, nb1, nlg, nlb, nw2, nb2,
      ng, nb_, fb2r)


def _edge_dist_stage(ssn, fw1r, fb1r, e):
    hd = fw1r.shape[1]
    be = 4096

    def body(ss_ref, fw1_ref, fb1_ref, out_ref):
        dist = jnp.sqrt(jnp.maximum(_unpack_cols(ss_ref[...], be), 1e-10))
        out_ref[...] = _silu(dist * fw1_ref[...] + fb1_ref[...])

    return pl.pallas_call(
        body,
        grid=(e // be,),
        in_specs=[
            pl.BlockSpec((be // 128, 128), lambda i: (i, 0)),
            _full(fw1r.shape), _full(fb1r.shape),
        ],
        out_specs=pl.BlockSpec((be, hd), lambda i: (i, 0)),
        out_shape=jax.ShapeDtypeStruct((e, hd), jnp.float32),
    )(ssn, fw1r, fb1r)


def _final_stage(hmid, fs, fw2):
    n, hd = hmid.shape
    nf = len(fs)
    bn = 2000

    def body(hm_ref, *refs):
        f_refs = refs[:nf]
        fw2_ref, out_ref = refs[nf:]
        s = sum(r[...] for r in f_refs[1:]) + f_refs[0][...]
        out_ref[...] = hm_ref[...] + 0.1 * _bdot(s, fw2_ref[...])

    return pl.pallas_call(
        body,
        grid=(n // bn,),
        in_specs=[pl.BlockSpec((bn, hd), lambda i: (i, 0))] * (1 + nf)
        + [_full(fw2.shape)],
        out_specs=pl.BlockSpec((bn, hd), lambda i: (i, 0)),
        out_shape=jax.ShapeDtypeStruct((n, hd), jnp.float32),
    )(hmid, *fs, fw2)


def kernel(h, pos, edge_attr, edge_index, ew1, eb1, elg, elb, ew2, eb2,
           nw1, nb1, nlg, nlb, nw2, nb2, ng, nb, cw1, cb1, cw2, aw, ab,
           fw1, fb1, fw2, fb2):
    n, hd = h.shape
    e = edge_index.shape[1]
    row = edge_index[0]
    col = edge_index[1]

    pos_pad = jnp.pad(pos, ((0, 0), (0, 16 - pos.shape[1])))
    r1 = lambda v: v.reshape(1, -1)
    tpb = (-(-n // _NS) + 7) // 8 * 8
    npad = tpb * _NS

    nslab = 2
    es = e // nslab
    es2 = -(-es // 4096) * 4096
    rows = [row[i * es:(i + 1) * es] for i in range(nslab)]
    cols = [col[i * es:(i + 1) * es] for i in range(nslab)]
    eas = [jnp.pad(edge_attr[i * es:(i + 1) * es], ((0, es2 - es), (0, 0)))
           for i in range(nslab)]

    gath = [_sc_gather_geom(h, pos_pad, rows[i], cols[i], es2)
            for i in range(nslab)]
    mlp = [_edge_mlp(
        gath[i][0], gath[i][1], gath[i][2], eas[i],
        ew1[:hd], ew1[hd:2 * hd], ew1[2 * hd:2 * hd + 1], ew1[2 * hd + 1:],
        r1(eb1), r1(elg), r1(elb), ew2, r1(eb2),
        aw.reshape(1, -1), ab.reshape(1, 1), cw1, r1(cb1), cw2.reshape(1, -1))
        for i in range(nslab)]
    scat = [_sc_scatter_edge(mlp[i][0], mlp[i][1], rows[i], cols[i],
                             pos_pad, npad) for i in range(nslab)]
    ps = [s[0][:n] for s in scat] + [s[0][npad:npad + n] for s in scat]
    qs = [s[1][:n] for s in scat] + [s[1][npad:npad + n] for s in scat]

    hmid, pn = _node_mlp(
        h, ps, qs, pos_pad, nw1[:hd], nw1[hd:], r1(nb1), r1(nlg),
        r1(nlb), nw2, r1(nb2), r1(ng), r1(nb), r1(fb2))

    ssn = [_sc_gather_geom_only(pn, rows[i], cols[i], es2)
           for i in range(nslab)]
    s_e = [_edge_dist_stage(ssn[i], fw1, r1(fb1), es2) for i in range(nslab)]
    scat2 = [_sc_scatter_add(s_e[i], rows[i], npad) for i in range(nslab)]
    fs = [s[:n] for s in scat2] + [s[npad:npad + n] for s in scat2]

    h_new = _final_stage(hmid, fs, fw2)
    pos_new = pn[:, :pos.shape[1]]
    return (h_new, pos_new)
